# baseline (device time: 551703 ns/iter reference)
import jax
import jax.numpy as jnp
from jax import lax
from jax.experimental import pallas as pl
from jax.experimental.pallas import tpu as pltpu

T = 2048
D = 4096
V_LOCAL = 8192
V_SLICE = V_LOCAL // 4
N_CHUNK = 1024
KBLK = 512
NK = D // KBLK
SUB = 256


def kernel(x, W, labels):
    labels2d = labels.reshape(T, 1)

    def x_index(n, k):
        return (0, jnp.minimum(k + n * NK, NK - 1))

    def w_index(n, k):
        q = lax.axis_index("x") * 2 + lax.axis_index("y")
        return (k, q * (V_SLICE // N_CHUNK) + n)

    def body(xw_ref, w_ref, lab_ref, out_ref, x16_ref, buf_a, buf_b,
             s_ref, ll_ref, comm_ref, send_sems, recv_sems):
        n = pl.program_id(0)
        k = pl.program_id(1)
        mx = lax.axis_index("x")
        my = lax.axis_index("y")
        mz = lax.axis_index("z")
        q = mx * 2 + my
        v0 = mz * V_LOCAL + q * V_SLICE

        partners = [
            (1 - mx, my, mz),
            (mx, 1 - my, mz),
            (mx, my, 1 - mz),
        ]

        @pl.when((n == 0) & (k == 0))
        def _():
            s_ref[:, :] = jnp.zeros((T, 1), jnp.float32)
            ll_ref[:, :] = jnp.zeros((T, 1), jnp.float32)
            barrier = pltpu.get_barrier_semaphore()
            for tgt in partners:
                pl.semaphore_signal(
                    barrier, inc=1,
                    device_id=tgt, device_id_type=pl.DeviceIdType.MESH,
                )
            pl.semaphore_wait(barrier, 3)

        def consume_sub(lg, col0):
            col = lax.broadcasted_iota(jnp.int32, (T, SUB), 1) + col0
            ll_s = jnp.sum(jnp.where(col == lab_ref[:, :], lg, 0.0),
                           axis=1, keepdims=True)
            s_s = jnp.sum(jnp.exp(lg), axis=1, keepdims=True)
            return s_s, ll_s

        def matmul_step(buf, kk):
            xs = x16_ref[:, kk * KBLK:(kk + 1) * KBLK]
            wv = w_ref[:, :].astype(jnp.bfloat16)
            d = jnp.dot(xs, wv, preferred_element_type=jnp.float32)
            if kk == 0:
                buf[:, :] = d
            else:
                buf[:, :] = buf[:, :] + d

        @pl.when(n == 0)
        def _():
            for kk in range(NK):
                @pl.when(k == kk)
                def _(kk=kk):
                    x16_ref[:, kk * KBLK:(kk + 1) * KBLK] = (
                        xw_ref[:, :].astype(jnp.bfloat16))
                    matmul_step(buf_a, kk)

        @pl.when(n == 1)
        def _():
            for kk in range(NK):
                @pl.when(k == kk)
                def _(kk=kk):
                    matmul_step(buf_b, kk)
                    if kk < N_CHUNK // SUB:
                        lg = buf_a[:, kk * SUB:(kk + 1) * SUB]
                        s_s, ll_s = consume_sub(lg, v0 + kk * SUB)
                        s_ref[:, :] += s_s
                        ll_ref[:, :] += ll_s

        @pl.when((n == 1) & (k == NK - 1))
        def _():
            s_cur = s_ref[:, :]
            ll_cur = ll_ref[:, :]
            for c in range(N_CHUNK // SUB):
                lg = buf_b[:, c * SUB:(c + 1) * SUB]
                s_s, ll_s = consume_sub(lg, v0 + N_CHUNK + c * SUB)
                s_cur = s_cur + s_s
                ll_cur = ll_cur + ll_s

            for j, tgt in enumerate(partners):
                comm_ref[0, :, 0:1] = s_cur
                comm_ref[0, :, 1:2] = ll_cur
                rdma = pltpu.make_async_remote_copy(
                    src_ref=comm_ref.at[0],
                    dst_ref=comm_ref.at[j + 1],
                    send_sem=send_sems.at[j],
                    recv_sem=recv_sems.at[j],
                    device_id=tgt,
                    device_id_type=pl.DeviceIdType.MESH,
                )
                rdma.start()
                rdma.wait()
                s_cur = s_cur + comm_ref[j + 1, :, 0:1]
                ll_cur = ll_cur + comm_ref[j + 1, :, 1:2]

            out_ref[:, :] = jnp.log(s_cur) - ll_cur

    out = pl.pallas_call(
        body,
        grid=(V_SLICE // N_CHUNK, NK),
        out_shape=jax.ShapeDtypeStruct((T, 1), jnp.float32),
        in_specs=[
            pl.BlockSpec((T, KBLK), x_index),
            pl.BlockSpec((KBLK, N_CHUNK), w_index),
            pl.BlockSpec((T, 1), lambda n, k: (0, 0)),
        ],
        out_specs=pl.BlockSpec((T, 1), lambda n, k: (0, 0)),
        scratch_shapes=[
            pltpu.VMEM((T, D), jnp.bfloat16),
            pltpu.VMEM((T, N_CHUNK), jnp.float32),
            pltpu.VMEM((T, N_CHUNK), jnp.float32),
            pltpu.VMEM((T, 1), jnp.float32),
            pltpu.VMEM((T, 1), jnp.float32),
            pltpu.VMEM((4, T, 2), jnp.float32),
            pltpu.SemaphoreType.DMA((3,)),
            pltpu.SemaphoreType.DMA((3,)),
        ],
        compiler_params=pltpu.CompilerParams(
            collective_id=0,
            dimension_semantics=("arbitrary", "arbitrary"),
            vmem_limit_bytes=100 * 1024 * 1024,
        ),
    )(x, W, labels2d)
    return out.reshape(T)


# device time: 238785 ns/iter; 2.3105x vs baseline; 2.3105x over previous
import jax
import jax.numpy as jnp
from jax import lax
from jax.experimental import pallas as pl
from jax.experimental.pallas import tpu as pltpu

T = 2048
D = 4096
V_LOCAL = 8192
V_SLICE = V_LOCAL // 4
N_CHUNK = 512
SUB = 128


def kernel(x, W, labels):
    labels2d = labels.reshape(T, 1)
    x16 = x.astype(jnp.bfloat16)
    q_out = lax.axis_index("x") * 2 + lax.axis_index("y")
    Wq16 = lax.dynamic_slice_in_dim(W, q_out * V_SLICE, V_SLICE, axis=1
                                    ).astype(jnp.bfloat16)

    def body(x_ref, w_ref, lab_ref, out_ref, buf_a, buf_b,
             s_ref, ll_ref, comm_ref, send_sems, recv_sems):
        j = pl.program_id(0)
        mx = lax.axis_index("x")
        my = lax.axis_index("y")
        mz = lax.axis_index("z")
        q = mx * 2 + my
        v0 = mz * V_LOCAL + q * V_SLICE

        partners = [
            (1 - mx, my, mz),
            (mx, 1 - my, mz),
            (mx, my, 1 - mz),
        ]

        @pl.when(j == 0)
        def _():
            s_ref[:, :] = jnp.zeros((T, 1), jnp.float32)
            ll_ref[:, :] = jnp.zeros((T, 1), jnp.float32)
            barrier = pltpu.get_barrier_semaphore()
            for tgt in partners:
                pl.semaphore_signal(
                    barrier, inc=1,
                    device_id=tgt, device_id_type=pl.DeviceIdType.MESH,
                )
            pl.semaphore_wait(barrier, 3)

        def consume_sub(lg, col0):
            col = lax.broadcasted_iota(jnp.int32, (T, SUB), 1) + col0
            ll_s = jnp.sum(jnp.where(col == lab_ref[:, :], lg, 0.0),
                           axis=1, keepdims=True)
            s_s = jnp.sum(jnp.exp(lg), axis=1, keepdims=True)
            return s_s, ll_s

        NCH = V_SLICE // N_CHUNK
        for jj in range(NCH):
            wbuf, rbuf = (buf_a, buf_b) if jj % 2 == 0 else (buf_b, buf_a)

            @pl.when(j == jj)
            def _(jj=jj, wbuf=wbuf, rbuf=rbuf):
                wbuf[:, :] = jnp.dot(x_ref[:, :], w_ref[:, :],
                                     preferred_element_type=jnp.float32)
                if jj > 0:
                    for c in range(N_CHUNK // SUB):
                        s_s, ll_s = consume_sub(
                            rbuf[:, c * SUB:(c + 1) * SUB],
                            v0 + (jj - 1) * N_CHUNK + c * SUB)
                        s_ref[:, :] += s_s
                        ll_ref[:, :] += ll_s

        last_buf = buf_a if (NCH - 1) % 2 == 0 else buf_b

        @pl.when(j == NCH - 1)
        def _():
            s_cur = s_ref[:, :]
            ll_cur = ll_ref[:, :]
            for c in range(N_CHUNK // SUB):
                s_s, ll_s = consume_sub(last_buf[:, c * SUB:(c + 1) * SUB],
                                        v0 + (NCH - 1) * N_CHUNK + c * SUB)
                s_cur = s_cur + s_s
                ll_cur = ll_cur + ll_s

            for k, tgt in enumerate(partners):
                comm_ref[0, :, 0:1] = s_cur
                comm_ref[0, :, 1:2] = ll_cur
                rdma = pltpu.make_async_remote_copy(
                    src_ref=comm_ref.at[0],
                    dst_ref=comm_ref.at[k + 1],
                    send_sem=send_sems.at[k],
                    recv_sem=recv_sems.at[k],
                    device_id=tgt,
                    device_id_type=pl.DeviceIdType.MESH,
                )
                rdma.start()
                rdma.wait()
                s_cur = s_cur + comm_ref[k + 1, :, 0:1]
                ll_cur = ll_cur + comm_ref[k + 1, :, 1:2]

            out_ref[:, :] = jnp.log(s_cur) - ll_cur

    out = pl.pallas_call(
        body,
        grid=(V_SLICE // N_CHUNK,),
        out_shape=jax.ShapeDtypeStruct((T, 1), jnp.float32),
        in_specs=[
            pl.BlockSpec((T, D), lambda j: (0, 0)),
            pl.BlockSpec((D, N_CHUNK), lambda j: (0, j)),
            pl.BlockSpec((T, 1), lambda j: (0, 0)),
        ],
        out_specs=pl.BlockSpec((T, 1), lambda j: (0, 0)),
        scratch_shapes=[
            pltpu.VMEM((T, N_CHUNK), jnp.float32),
            pltpu.VMEM((T, N_CHUNK), jnp.float32),
            pltpu.VMEM((T, 1), jnp.float32),
            pltpu.VMEM((T, 1), jnp.float32),
            pltpu.VMEM((4, T, 2), jnp.float32),
            pltpu.SemaphoreType.DMA((3,)),
            pltpu.SemaphoreType.DMA((3,)),
        ],
        compiler_params=pltpu.CompilerParams(
            collective_id=0,
            dimension_semantics=("arbitrary",),
            vmem_limit_bytes=100 * 1024 * 1024,
        ),
    )(x16, Wq16, labels2d)
    return out.reshape(T)


# device time: 118092 ns/iter; 4.6718x vs baseline; 2.0220x over previous
import jax
import jax.numpy as jnp
from jax import lax
from jax.experimental import pallas as pl
from jax.experimental.pallas import tpu as pltpu

T = 2048
D = 4096
V_LOCAL = 8192
V_SLICE = V_LOCAL // 4
BLK_V = 256
NBLK = V_SLICE // BLK_V
assert NBLK % 2 == 0


def kernel(x, W, labels):
    labels2d = labels.reshape(T, 1)
    x16 = x.astype(jnp.bfloat16)
    nblk_slice = V_SLICE // BLK_V

    def w_index(j):
        q = lax.axis_index("x") * 2 + lax.axis_index("y")
        return (0, q * nblk_slice + j)

    def body(x_ref, w_ref, lab_ref, out_ref, buf_a, buf_b, s_ref, ll_ref,
             comm_ref, send_sems, recv_sems):
        j = pl.program_id(0)
        mx = lax.axis_index("x")
        my = lax.axis_index("y")
        mz = lax.axis_index("z")
        q = mx * 2 + my
        v0 = mz * V_LOCAL + q * V_SLICE

        partners = [
            (1 - mx, my, mz),
            (mx, 1 - my, mz),
            (mx, my, 1 - mz),
        ]

        @pl.when(j == 0)
        def _():
            s_ref[:, :] = jnp.zeros((T, 1), jnp.float32)
            ll_ref[:, :] = jnp.zeros((T, 1), jnp.float32)
            barrier = pltpu.get_barrier_semaphore()
            for tgt in partners:
                pl.semaphore_signal(
                    barrier, inc=1,
                    device_id=tgt, device_id_type=pl.DeviceIdType.MESH,
                )
            pl.semaphore_wait(barrier, 3)

        def consume(lg, blk_idx):
            col = (lax.broadcasted_iota(jnp.int32, (T, BLK_V), 1)
                   + blk_idx * BLK_V + v0)
            ll_blk = jnp.sum(jnp.where(col == lab_ref[:, :], lg, 0.0),
                             axis=1, keepdims=True)
            s_blk = jnp.sum(jnp.exp(lg), axis=1, keepdims=True)
            return s_blk, ll_blk

        def do_step(wbuf, rbuf):
            wbuf[:, :] = jnp.dot(x_ref[:, :],
                                 w_ref[:, :].astype(jnp.bfloat16),
                                 preferred_element_type=jnp.float32)
            s_blk, ll_blk = consume(rbuf[:, :], j - 1)

            @pl.when(j > 0)
            def _():
                s_ref[:, :] += s_blk
                ll_ref[:, :] += ll_blk

        @pl.when(j % 2 == 0)
        def _():
            do_step(buf_a, buf_b)

        @pl.when(j % 2 == 1)
        def _():
            do_step(buf_b, buf_a)

        @pl.when(j == NBLK - 1)
        def _():
            s_blk, ll_blk = consume(buf_b[:, :], NBLK - 1)
            s_cur = s_ref[:, :] + s_blk
            ll_cur = ll_ref[:, :] + ll_blk

            for k, tgt in enumerate(partners):
                comm_ref[0, :, 0:1] = s_cur
                comm_ref[0, :, 1:2] = ll_cur
                rdma = pltpu.make_async_remote_copy(
                    src_ref=comm_ref.at[0],
                    dst_ref=comm_ref.at[k + 1],
                    send_sem=send_sems.at[k],
                    recv_sem=recv_sems.at[k],
                    device_id=tgt,
                    device_id_type=pl.DeviceIdType.MESH,
                )
                rdma.start()
                rdma.wait()
                s_cur = s_cur + comm_ref[k + 1, :, 0:1]
                ll_cur = ll_cur + comm_ref[k + 1, :, 1:2]

            out_ref[:, :] = jnp.log(s_cur) - ll_cur

    out = pl.pallas_call(
        body,
        grid=(NBLK,),
        out_shape=jax.ShapeDtypeStruct((T, 1), jnp.float32),
        in_specs=[
            pl.BlockSpec((T, D), lambda j: (0, 0)),
            pl.BlockSpec((D, BLK_V), w_index),
            pl.BlockSpec((T, 1), lambda j: (0, 0)),
        ],
        out_specs=pl.BlockSpec((T, 1), lambda j: (0, 0)),
        scratch_shapes=[
            pltpu.VMEM((T, BLK_V), jnp.float32),
            pltpu.VMEM((T, BLK_V), jnp.float32),
            pltpu.VMEM((T, 1), jnp.float32),
            pltpu.VMEM((T, 1), jnp.float32),
            pltpu.VMEM((4, T, 2), jnp.float32),
            pltpu.SemaphoreType.DMA((3,)),
            pltpu.SemaphoreType.DMA((3,)),
        ],
        compiler_params=pltpu.CompilerParams(
            collective_id=0,
            dimension_semantics=("arbitrary",),
            vmem_limit_bytes=100 * 1024 * 1024,
        ),
    )(x16, W, labels2d)
    return out.reshape(T)
